# baseline (device time: 384811 ns/iter reference)
import jax
import jax.numpy as jnp
from jax import lax
from jax.experimental import pallas as pl
from jax.experimental.pallas import tpu as pltpu

N_DEV = 4
M_PER = 2048
K_BLK = 2048
N_TOT = 4096
NTW = 256
NTS = N_TOT // NTW
WFR = 128
WF = K_BLK // WFR
N_STEPS = N_DEV * NTS


def kernel(x, w_mat):
    x16 = x.astype(jnp.bfloat16)

    def body(x_ref, w_ref, o_ref, o16, wblk, xsl, wstage, fstage,
             lsem, wsems, fsems, send_sems, recv_sems, credit_sems):
        j = pl.program_id(0)
        nt = pl.program_id(1)
        s = j * NTS + nt
        me = lax.axis_index("i")

        def src_of(jj):
            return lax.rem(me + lax.bitwise_xor(jj, jj // 2), N_DEV)

        def mk_send(off, ssem, xslot, rsem):
            t = lax.rem(me + off, N_DEV)
            return pltpu.make_async_remote_copy(
                src_ref=x_ref.at[pl.ds(t * M_PER, M_PER), :],
                dst_ref=xsl.at[xslot],
                send_sem=send_sems.at[ssem],
                recv_sem=recv_sems.at[rsem],
                device_id=(t,),
                device_id_type=pl.DeviceIdType.MESH,
            )

        def mk_send_half(off, mh, ssem, rsem):
            t = lax.rem(me + off, N_DEV)
            return pltpu.make_async_remote_copy(
                src_ref=x_ref.at[pl.ds(t * M_PER + mh * (M_PER // 2),
                                       M_PER // 2), :],
                dst_ref=xsl.at[1, pl.ds(mh * (M_PER // 2), M_PER // 2), :],
                send_sem=send_sems.at[ssem],
                recv_sem=recv_sems.at[rsem],
                device_id=(t,),
                device_id_type=pl.DeviceIdType.MESH,
            )

        def mk_recv(xslot, rsem):
            return pltpu.make_async_remote_copy(
                src_ref=xsl.at[xslot],
                dst_ref=xsl.at[xslot],
                send_sem=send_sems.at[0],
                recv_sem=recv_sems.at[rsem],
                device_id=(0,),
                device_id_type=pl.DeviceIdType.MESH,
            )

        def mk_recv_half(mh, rsem):
            return pltpu.make_async_remote_copy(
                src_ref=xsl.at[1, pl.ds(mh * (M_PER // 2), M_PER // 2), :],
                dst_ref=xsl.at[1, pl.ds(mh * (M_PER // 2), M_PER // 2), :],
                send_sem=send_sems.at[0],
                recv_sem=recv_sems.at[rsem],
                device_id=(0,),
                device_id_type=pl.DeviceIdType.MESH,
            )

        def mk_stage(jj, c, slot):
            return pltpu.make_async_copy(
                w_ref.at[pl.ds(src_of(jj) * K_BLK + c * WFR, WFR), :],
                wstage.at[slot],
                wsems.at[slot],
            )

        def mk_fstore(tt, u):
            sl = lax.rem(tt, 2)
            mh = tt // 8
            t_idx = lax.rem(tt, 8) * 2 + u
            return pltpu.make_async_copy(
                fstage.at[sl, pl.ds(u * (M_PER // 2), M_PER // 2), :],
                o_ref.at[pl.ds(mh * (M_PER // 2), M_PER // 2),
                         pl.ds(t_idx * NTW, NTW)],
                fsems.at[sl],
            )

        SENDS = {
            "j1": (3, 0, 1, 1),
            "j2": (1, 1, 0, 2),
        }

        @pl.when(s == 0)
        def _():
            barrier_sem = pltpu.get_barrier_semaphore()
            for off in (1, 2, 3):
                t = lax.rem(me + off, N_DEV)
                pl.semaphore_signal(
                    barrier_sem, inc=1,
                    device_id=(t,), device_id_type=pl.DeviceIdType.MESH,
                )
            pl.semaphore_wait(barrier_sem, 3)
            mk_send(*SENDS["j1"]).start()
            cp = pltpu.make_async_copy(
                x_ref.at[pl.ds(me * M_PER, M_PER), :], xsl.at[0], lsem
            )
            cp.start()

        @pl.when(nt == 0)
        def _():
            @pl.when(j == 1)
            def _():
                pl.semaphore_signal(
                    credit_sems.at[0], inc=1,
                    device_id=(lax.rem(me + 3, N_DEV),),
                    device_id_type=pl.DeviceIdType.MESH,
                )
                pl.semaphore_wait(credit_sems.at[0], 1)
                mk_send(*SENDS["j2"]).start()

            @pl.when(j == 2)
            def _():
                pl.semaphore_signal(
                    credit_sems.at[1], inc=1,
                    device_id=(lax.rem(me + 2, N_DEV),),
                    device_id_type=pl.DeviceIdType.MESH,
                )
                pl.semaphore_wait(credit_sems.at[1], 1)
                mk_send_half(2, 0, 2, 3).start()
                mk_send_half(2, 1, 3, 4).start()

            mk_stage(j, 0, 0).start()
            for c in range(WF):
                if c + 1 < WF:
                    mk_stage(j, c + 1, (c + 1) % 2).start()
                mk_stage(j, c, c % 2).wait()
                wblk[c * WFR:(c + 1) * WFR, :] = (
                    wstage[c % 2].astype(jnp.bfloat16)
                )

            @pl.when(j == 0)
            def _():
                pltpu.make_async_copy(
                    x_ref.at[pl.ds(me * M_PER, M_PER), :], xsl.at[0], lsem
                ).wait()

            @pl.when(j == 1)
            def _():
                mk_recv(1, 1).wait_recv()

            @pl.when(j == 2)
            def _():
                mk_recv(0, 2).wait_recv()

        @pl.when(j < 3)
        def _():
            cols = pl.ds(nt * NTW, NTW)
            val = jnp.dot(
                xsl[lax.rem(j, 2)], wblk[:, cols],
                preferred_element_type=jnp.float32,
            )

            @pl.when(j == 0)
            def _():
                o16[:, cols] = val.astype(jnp.bfloat16)

            @pl.when(j > 0)
            def _():
                o16[:, cols] = (
                    o16[:, cols].astype(jnp.float32) + val
                ).astype(jnp.bfloat16)

        @pl.when(j == N_DEV - 1)
        def _():
            @pl.when(nt == 0)
            def _():
                mk_recv_half(0, 3).wait_recv()

            @pl.when(nt == 8)
            def _():
                mk_recv_half(1, 4).wait_recv()

            @pl.when(nt >= 2)
            def _():
                mk_fstore(nt - 2, 0).wait()
                mk_fstore(nt - 2, 1).wait()

            mh = nt // 8
            rows = pl.ds(mh * (M_PER // 2), M_PER // 2)
            sl = lax.rem(nt, 2)
            xh = xsl[1, rows, :]
            for u in (0, 1):
                t_idx = lax.rem(nt, 8) * 2 + u
                cols_u = pl.ds(t_idx * NTW, NTW)
                val_u = jnp.dot(
                    xh, wblk[:, cols_u], preferred_element_type=jnp.float32
                )
                fstage[sl, pl.ds(u * (M_PER // 2), M_PER // 2), :] = (
                    jnp.maximum(
                        o16[rows, cols_u].astype(jnp.float32) + val_u, 0.0
                    )
                )
                mk_fstore(nt, u).start()

        @pl.when(s == N_STEPS - 1)
        def _():
            for tt in (NTS - 2, NTS - 1):
                mk_fstore(tt, 0).wait()
                mk_fstore(tt, 1).wait()
            for key in ("j1", "j2"):
                mk_send(*SENDS[key]).wait_send()
            mk_send_half(2, 0, 2, 3).wait_send()
            mk_send_half(2, 1, 3, 4).wait_send()

    return pl.pallas_call(
        body,
        grid=(N_DEV, NTS),
        out_shape=jax.ShapeDtypeStruct((M_PER, N_TOT), jnp.float32),
        in_specs=[
            pl.BlockSpec(memory_space=pl.ANY),
            pl.BlockSpec(memory_space=pl.ANY),
        ],
        out_specs=pl.BlockSpec(memory_space=pl.ANY),
        scratch_shapes=[
            pltpu.VMEM((M_PER, N_TOT), jnp.bfloat16),
            pltpu.VMEM((K_BLK, N_TOT), jnp.bfloat16),
            pltpu.VMEM((2, M_PER, K_BLK), jnp.bfloat16),
            pltpu.VMEM((2, WFR, N_TOT), jnp.float32),
            pltpu.VMEM((2, M_PER, NTW), jnp.float32),
            pltpu.SemaphoreType.DMA,
            pltpu.SemaphoreType.DMA((2,)),
            pltpu.SemaphoreType.DMA((2,)),
            pltpu.SemaphoreType.DMA((4,)),
            pltpu.SemaphoreType.DMA((5,)),
            pltpu.SemaphoreType.REGULAR((2,)),
        ],
        compiler_params=pltpu.CompilerParams(
            collective_id=0,
            vmem_limit_bytes=63 * 1024 * 1024,
        ),
    )(x16, w_mat)
